# baseline (speedup 1.0000x reference)
import functools

import numpy as np
import jax
import jax.numpy as jnp
from jax.experimental import pallas as pl
from jax.experimental.pallas import tpu as pltpu

MHN = 4
NHEADS = 4
CIN = 128
COUT = 128
B = 64
S = 256
A = 5
K = 20
ALPHAS = (0.05, 0.06, 0.08, 0.1, 0.12)


def _qpos(q):
    loc = q * (K - 1)
    f = int(np.floor(loc))
    return f, float(loc - f)


_LOW = [_qpos(a / 2.0) for a in ALPHAS]
_HIGH = [_qpos(1.0 - a + a / 2.0) for a in ALPHAS]


def _body(x_ref, y_ref, w0, c0, w1, c1, w2, c2, w3, c3, wq, wk,
          ylow_ref, yhigh_ref, sc_ref):
    b = pl.program_id(1)
    x = x_ref[0, 0]
    yrow = y_ref[0, 0]

    h = jnp.maximum(jnp.dot(x, w0[0], preferred_element_type=jnp.float32) + c0[0], 0.0)
    h = jnp.maximum(jnp.dot(h, w1[0], preferred_element_type=jnp.float32) + c1[0], 0.0)
    h = jnp.maximum(jnp.dot(h, w2[0], preferred_element_type=jnp.float32) + c2[0], 0.0)
    enc = jnp.dot(h, w3[0], preferred_element_type=jnp.float32) + c3[0]
    q = jnp.dot(enc, wq[0], preferred_element_type=jnp.float32)
    k = jnp.dot(enc, wk[0], preferred_element_type=jnp.float32)

    SW = NHEADS * S
    ybc = jnp.broadcast_to(yrow.reshape(S, 1), (S, SW))
    yw = jnp.concatenate([yrow] * NHEADS, axis=1)
    inf = jnp.float32(np.inf)

    cur = jnp.concatenate([
        jax.lax.dot_general(k[:, hh * COUT:(hh + 1) * COUT],
                            q[:, hh * COUT:(hh + 1) * COUT],
                            (((1,), (1,)), ((), ())),
                            preferred_element_type=jnp.float32)
        for hh in range(NHEADS)], axis=1)
    for _ in range(K):
        mx = jnp.max(cur, axis=0, keepdims=True)
        cur = jnp.where(cur == mx, -inf, cur)
    sel = cur == -inf
    ylo = jnp.where(sel, ybc, inf)
    yhi = jnp.where(sel, ybc, -inf)
    m1 = jnp.min(ylo, axis=0, keepdims=True)
    ylo = jnp.where(ylo == m1, inf, ylo)
    m2 = jnp.min(ylo, axis=0, keepdims=True)
    ylo = jnp.where(ylo == m2, inf, ylo)
    m3 = jnp.min(ylo, axis=0, keepdims=True)
    b1 = jnp.max(yhi, axis=0, keepdims=True)
    yhi = jnp.where(yhi == b1, -inf, yhi)
    b2 = jnp.max(yhi, axis=0, keepdims=True)
    yhi = jnp.where(yhi == b2, -inf, yhi)
    b3 = jnp.max(yhi, axis=0, keepdims=True)
    lows = []
    for f, g in _LOW:
        lo_v, hi_v = (m1, m2) if f == 0 else (m2, m3)
        lows.append(lo_v * jnp.float32(1.0 - g) + hi_v * jnp.float32(g))
    highs = []
    for f, g in _HIGH:
        lo_v, hi_v = (b2, b1) if f == 18 else (b3, b2)
        highs.append(lo_v * jnp.float32(1.0 - g) + hi_v * jnp.float32(g))
    qlow = jnp.concatenate(lows, axis=0)
    qhigh = jnp.concatenate(highs, axis=0)
    for hh in range(NHEADS):
        ylow_ref[0, 0, hh] = qlow[:, hh * S:(hh + 1) * S]
        yhigh_ref[0, 0, hh] = qhigh[:, hh * S:(hh + 1) * S]
    total = jnp.float32(0.0)
    for ai, a in enumerate(ALPHAS):
        ql = lows[ai]
        qh2 = highs[ai]
        c = jnp.float32(2.0 / a)
        pen = (jnp.abs(qh2 - ql)
               + jnp.where(yw < ql, (ql - yw) * c, 0.0)
               + jnp.where(yw > qh2, (yw - qh2) * c, 0.0))
        total = total + jnp.sum(pen)

    @pl.when(b == 0)
    def _init():
        sc_ref[...] = jnp.zeros_like(sc_ref)

    sc_ref[...] += total * jnp.float32(1.0 / (NHEADS * A * B * S))


def kernel(X_d, Y_d,
           h0_W0, h0_b0, h0_W1, h0_b1, h0_W2, h0_b2, h0_W3, h0_b3, hop0_Wq, hop0_Wk,
           h1_W0, h1_b0, h1_W1, h1_b1, h1_W2, h1_b2, h1_W3, h1_b3, hop1_Wq, hop1_Wk,
           h2_W0, h2_b0, h2_W1, h2_b1, h2_W2, h2_b2, h2_W3, h2_b3, hop2_Wq, hop2_Wk,
           h3_W0, h3_b0, h3_W1, h3_b1, h3_W2, h3_b2, h3_W3, h3_b3, hop3_Wq, hop3_Wk,
           train=0):
    per_model = [
        [h0_W0, h0_b0, h0_W1, h0_b1, h0_W2, h0_b2, h0_W3, h0_b3, hop0_Wq, hop0_Wk],
        [h1_W0, h1_b0, h1_W1, h1_b1, h1_W2, h1_b2, h1_W3, h1_b3, hop1_Wq, hop1_Wk],
        [h2_W0, h2_b0, h2_W1, h2_b1, h2_W2, h2_b2, h2_W3, h2_b3, hop2_Wq, hop2_Wk],
        [h3_W0, h3_b0, h3_W1, h3_b1, h3_W2, h3_b2, h3_W3, h3_b3, hop3_Wq, hop3_Wk],
    ]
    stacked = []
    for j in range(10):
        arrs = [per_model[i][j] for i in range(MHN)]
        st = jnp.stack(arrs)
        if st.ndim == 2:
            st = st[:, None, :]
        stacked.append(st)

    Y2 = Y_d[:, :, :, 0].reshape(B, MHN, 1, S)

    in_specs = [
        pl.BlockSpec((1, 1, S, CIN), lambda i, b: (b, i, 0, 0)),
        pl.BlockSpec((1, 1, 1, S), lambda i, b: (b, i, 0, 0)),
    ]
    for st in stacked:
        in_specs.append(
            pl.BlockSpec((1,) + st.shape[1:], lambda i, b: (i, 0, 0)))

    out_shape = [
        jax.ShapeDtypeStruct((B, MHN, NHEADS, A, S), jnp.float32),
        jax.ShapeDtypeStruct((B, MHN, NHEADS, A, S), jnp.float32),
        jax.ShapeDtypeStruct((MHN, 8, 128), jnp.float32),
    ]
    out_specs = [
        pl.BlockSpec((1, 1, NHEADS, A, S), lambda i, b: (b, i, 0, 0, 0)),
        pl.BlockSpec((1, 1, NHEADS, A, S), lambda i, b: (b, i, 0, 0, 0)),
        pl.BlockSpec((1, 8, 128), lambda i, b: (i, 0, 0)),
    ]

    ylow_k, yhigh_k, sc_k = pl.pallas_call(
        _body,
        grid=(MHN, B),
        in_specs=in_specs,
        out_specs=out_specs,
        out_shape=out_shape,
        compiler_params=pltpu.CompilerParams(
            dimension_semantics=("arbitrary", "arbitrary")),
    )(X_d, Y2, *stacked)

    scores = sc_k[:, 0, 0] + jnp.asarray(train, jnp.float32) * 0.0
    y = jnp.transpose(Y_d[:, :, :, 0], (0, 2, 1))
    y_low = jnp.transpose(ylow_k, (2, 0, 3, 4, 1))
    y_high = jnp.transpose(yhigh_k, (2, 0, 3, 4, 1))
    return (scores, y, y_low, y_high)


# speedup vs baseline: 1.0449x; 1.0449x over previous
import functools

import numpy as np
import jax
import jax.numpy as jnp
from jax.experimental import pallas as pl
from jax.experimental.pallas import tpu as pltpu

MHN = 4
NHEADS = 4
CIN = 128
COUT = 128
B = 64
S = 256
A = 5
K = 20
ALPHAS = (0.05, 0.06, 0.08, 0.1, 0.12)


def _qpos(q):
    loc = q * (K - 1)
    f = int(np.floor(loc))
    return f, float(loc - f)


_LOW = [_qpos(a / 2.0) for a in ALPHAS]
_HIGH = [_qpos(1.0 - a + a / 2.0) for a in ALPHAS]


def _body(x_ref, y_ref, w0, c0, w1, c1, w2, c2, w3, c3, wq, wk,
          ylow_ref, yhigh_ref, sc_ref):
    b = pl.program_id(1)
    x = x_ref[0, 0]
    yrow = y_ref[0, 0]

    h = jnp.maximum(jnp.dot(x, w0[0], preferred_element_type=jnp.float32) + c0[0], 0.0)
    h = jnp.maximum(jnp.dot(h, w1[0], preferred_element_type=jnp.float32) + c1[0], 0.0)
    h = jnp.maximum(jnp.dot(h, w2[0], preferred_element_type=jnp.float32) + c2[0], 0.0)
    enc = jnp.dot(h, w3[0], preferred_element_type=jnp.float32) + c3[0]
    q = jnp.dot(enc, wq[0], preferred_element_type=jnp.float32)
    k = jnp.dot(enc, wk[0], preferred_element_type=jnp.float32)

    ybc = jnp.broadcast_to(yrow.reshape(S, 1), (S, S))
    inf = jnp.float32(np.inf)

    curs = [jax.lax.dot_general(k[:, hh * COUT:(hh + 1) * COUT],
                                q[:, hh * COUT:(hh + 1) * COUT],
                                (((1,), (1,)), ((), ())),
                                preferred_element_type=jnp.float32)
            for hh in range(NHEADS)]
    for _ in range(K):
        mxs = [jnp.max(c, axis=0, keepdims=True) for c in curs]
        curs = [jnp.where(c == m, -inf, c) for c, m in zip(curs, mxs)]
    sels = [c == -inf for c in curs]
    ylos = [jnp.where(s, ybc, inf) for s in sels]
    yhis = [jnp.where(s, ybc, -inf) for s in sels]
    m1s = [jnp.min(v, axis=0, keepdims=True) for v in ylos]
    ylos = [jnp.where(v == m, inf, v) for v, m in zip(ylos, m1s)]
    m2s = [jnp.min(v, axis=0, keepdims=True) for v in ylos]
    ylos = [jnp.where(v == m, inf, v) for v, m in zip(ylos, m2s)]
    m3s = [jnp.min(v, axis=0, keepdims=True) for v in ylos]
    b1s = [jnp.max(v, axis=0, keepdims=True) for v in yhis]
    yhis = [jnp.where(v == m, -inf, v) for v, m in zip(yhis, b1s)]
    b2s = [jnp.max(v, axis=0, keepdims=True) for v in yhis]
    yhis = [jnp.where(v == m, -inf, v) for v, m in zip(yhis, b2s)]
    b3s = [jnp.max(v, axis=0, keepdims=True) for v in yhis]

    total = jnp.float32(0.0)
    for hh in range(NHEADS):
        m1, m2, m3 = m1s[hh], m2s[hh], m3s[hh]
        b1, b2, b3 = b1s[hh], b2s[hh], b3s[hh]
        lows = []
        for f, g in _LOW:
            lo_v, hi_v = (m1, m2) if f == 0 else (m2, m3)
            lows.append(lo_v * jnp.float32(1.0 - g) + hi_v * jnp.float32(g))
        highs = []
        for f, g in _HIGH:
            lo_v, hi_v = (b2, b1) if f == 18 else (b3, b2)
            highs.append(lo_v * jnp.float32(1.0 - g) + hi_v * jnp.float32(g))
        ylow_ref[0, 0, hh] = jnp.concatenate(lows, axis=0)
        yhigh_ref[0, 0, hh] = jnp.concatenate(highs, axis=0)
        for ai, a in enumerate(ALPHAS):
            ql = lows[ai]
            qh2 = highs[ai]
            c = jnp.float32(2.0 / a)
            pen = (jnp.abs(qh2 - ql)
                   + jnp.where(yrow < ql, (ql - yrow) * c, 0.0)
                   + jnp.where(yrow > qh2, (yrow - qh2) * c, 0.0))
            total = total + jnp.sum(pen)

    @pl.when(b == 0)
    def _init():
        sc_ref[...] = jnp.zeros_like(sc_ref)

    sc_ref[...] += total * jnp.float32(1.0 / (NHEADS * A * B * S))


def kernel(X_d, Y_d,
           h0_W0, h0_b0, h0_W1, h0_b1, h0_W2, h0_b2, h0_W3, h0_b3, hop0_Wq, hop0_Wk,
           h1_W0, h1_b0, h1_W1, h1_b1, h1_W2, h1_b2, h1_W3, h1_b3, hop1_Wq, hop1_Wk,
           h2_W0, h2_b0, h2_W1, h2_b1, h2_W2, h2_b2, h2_W3, h2_b3, hop2_Wq, hop2_Wk,
           h3_W0, h3_b0, h3_W1, h3_b1, h3_W2, h3_b2, h3_W3, h3_b3, hop3_Wq, hop3_Wk,
           train=0):
    per_model = [
        [h0_W0, h0_b0, h0_W1, h0_b1, h0_W2, h0_b2, h0_W3, h0_b3, hop0_Wq, hop0_Wk],
        [h1_W0, h1_b0, h1_W1, h1_b1, h1_W2, h1_b2, h1_W3, h1_b3, hop1_Wq, hop1_Wk],
        [h2_W0, h2_b0, h2_W1, h2_b1, h2_W2, h2_b2, h2_W3, h2_b3, hop2_Wq, hop2_Wk],
        [h3_W0, h3_b0, h3_W1, h3_b1, h3_W2, h3_b2, h3_W3, h3_b3, hop3_Wq, hop3_Wk],
    ]
    stacked = []
    for j in range(10):
        arrs = [per_model[i][j] for i in range(MHN)]
        st = jnp.stack(arrs)
        if st.ndim == 2:
            st = st[:, None, :]
        stacked.append(st)

    Y2 = Y_d[:, :, :, 0].reshape(B, MHN, 1, S)

    in_specs = [
        pl.BlockSpec((1, 1, S, CIN), lambda i, b: (b, i, 0, 0)),
        pl.BlockSpec((1, 1, 1, S), lambda i, b: (b, i, 0, 0)),
    ]
    for st in stacked:
        in_specs.append(
            pl.BlockSpec((1,) + st.shape[1:], lambda i, b: (i, 0, 0)))

    out_shape = [
        jax.ShapeDtypeStruct((B, MHN, NHEADS, A, S), jnp.float32),
        jax.ShapeDtypeStruct((B, MHN, NHEADS, A, S), jnp.float32),
        jax.ShapeDtypeStruct((MHN, 8, 128), jnp.float32),
    ]
    out_specs = [
        pl.BlockSpec((1, 1, NHEADS, A, S), lambda i, b: (b, i, 0, 0, 0)),
        pl.BlockSpec((1, 1, NHEADS, A, S), lambda i, b: (b, i, 0, 0, 0)),
        pl.BlockSpec((1, 8, 128), lambda i, b: (i, 0, 0)),
    ]

    ylow_k, yhigh_k, sc_k = pl.pallas_call(
        _body,
        grid=(MHN, B),
        in_specs=in_specs,
        out_specs=out_specs,
        out_shape=out_shape,
        compiler_params=pltpu.CompilerParams(
            dimension_semantics=("arbitrary", "arbitrary")),
    )(X_d, Y2, *stacked)

    scores = sc_k[:, 0, 0] + jnp.asarray(train, jnp.float32) * 0.0
    y = jnp.transpose(Y_d[:, :, :, 0], (0, 2, 1))
    y_low = jnp.transpose(ylow_k, (2, 0, 3, 4, 1))
    y_high = jnp.transpose(yhigh_k, (2, 0, 3, 4, 1))
    return (scores, y, y_low, y_high)


# speedup vs baseline: 1.2477x; 1.1941x over previous
import functools

import numpy as np
import jax
import jax.numpy as jnp
from jax.experimental import pallas as pl
from jax.experimental.pallas import tpu as pltpu

MHN = 4
NHEADS = 4
CIN = 128
COUT = 128
B = 64
S = 256
A = 5
K = 20
ALPHAS = (0.05, 0.06, 0.08, 0.1, 0.12)


def _qpos(q):
    loc = q * (K - 1)
    f = int(np.floor(loc))
    return f, float(loc - f)


_LOW = [_qpos(a / 2.0) for a in ALPHAS]
_HIGH = [_qpos(1.0 - a + a / 2.0) for a in ALPHAS]


NB = 4


def _body(x_ref, y_ref, w0, c0, w1, c1, w2, c2, w3, c3, wq, wk,
          ylow_ref, yhigh_ref, sc_ref):
    bb = pl.program_id(1)
    xall = x_ref[:, 0].reshape(NB * S, CIN)

    h = jnp.maximum(jnp.dot(xall, w0[0], preferred_element_type=jnp.float32) + c0[0], 0.0)
    h = jnp.maximum(jnp.dot(h, w1[0], preferred_element_type=jnp.float32) + c1[0], 0.0)
    h = jnp.maximum(jnp.dot(h, w2[0], preferred_element_type=jnp.float32) + c2[0], 0.0)
    enc = jnp.dot(h, w3[0], preferred_element_type=jnp.float32) + c3[0]
    q = jnp.dot(enc, wq[0], preferred_element_type=jnp.float32)
    k = jnp.dot(enc, wk[0], preferred_element_type=jnp.float32)

    inf = jnp.float32(np.inf)
    total = jnp.float32(0.0)

    for j in range(NB):
        yrow = y_ref[j, 0]
        ybc = jnp.broadcast_to(yrow.reshape(S, 1), (S, S))
        qj = q[j * S:(j + 1) * S]
        kj = k[j * S:(j + 1) * S]
        curs = [jax.lax.dot_general(kj[:, hh * COUT:(hh + 1) * COUT],
                                    qj[:, hh * COUT:(hh + 1) * COUT],
                                    (((1,), (1,)), ((), ())),
                                    preferred_element_type=jnp.float32)
                for hh in range(NHEADS)]
        for _ in range(K):
            mxs = [jnp.max(c, axis=0, keepdims=True) for c in curs]
            curs = [jnp.where(c == m, -inf, c) for c, m in zip(curs, mxs)]
        sels = [c == -inf for c in curs]
        ylos = [jnp.where(s, ybc, inf) for s in sels]
        yhis = [jnp.where(s, ybc, -inf) for s in sels]
        m1s = [jnp.min(v, axis=0, keepdims=True) for v in ylos]
        ylos = [jnp.where(v == m, inf, v) for v, m in zip(ylos, m1s)]
        m2s = [jnp.min(v, axis=0, keepdims=True) for v in ylos]
        ylos = [jnp.where(v == m, inf, v) for v, m in zip(ylos, m2s)]
        m3s = [jnp.min(v, axis=0, keepdims=True) for v in ylos]
        b1s = [jnp.max(v, axis=0, keepdims=True) for v in yhis]
        yhis = [jnp.where(v == m, -inf, v) for v, m in zip(yhis, b1s)]
        b2s = [jnp.max(v, axis=0, keepdims=True) for v in yhis]
        yhis = [jnp.where(v == m, -inf, v) for v, m in zip(yhis, b2s)]
        b3s = [jnp.max(v, axis=0, keepdims=True) for v in yhis]

        for hh in range(NHEADS):
            m1, m2, m3 = m1s[hh], m2s[hh], m3s[hh]
            b1, b2, b3 = b1s[hh], b2s[hh], b3s[hh]
            lows = []
            for f, g in _LOW:
                lo_v, hi_v = (m1, m2) if f == 0 else (m2, m3)
                lows.append(lo_v * jnp.float32(1.0 - g) + hi_v * jnp.float32(g))
            highs = []
            for f, g in _HIGH:
                lo_v, hi_v = (b2, b1) if f == 18 else (b3, b2)
                highs.append(lo_v * jnp.float32(1.0 - g) + hi_v * jnp.float32(g))
            ylow_ref[j, 0, hh] = jnp.concatenate(lows, axis=0)
            yhigh_ref[j, 0, hh] = jnp.concatenate(highs, axis=0)
            for ai, a in enumerate(ALPHAS):
                ql = lows[ai]
                qh2 = highs[ai]
                c = jnp.float32(2.0 / a)
                pen = (jnp.abs(qh2 - ql)
                       + jnp.where(yrow < ql, (ql - yrow) * c, 0.0)
                       + jnp.where(yrow > qh2, (yrow - qh2) * c, 0.0))
                total = total + jnp.sum(pen)

    @pl.when(bb == 0)
    def _init():
        sc_ref[...] = jnp.zeros_like(sc_ref)

    sc_ref[...] += total * jnp.float32(1.0 / (NHEADS * A * B * S))


def kernel(X_d, Y_d,
           h0_W0, h0_b0, h0_W1, h0_b1, h0_W2, h0_b2, h0_W3, h0_b3, hop0_Wq, hop0_Wk,
           h1_W0, h1_b0, h1_W1, h1_b1, h1_W2, h1_b2, h1_W3, h1_b3, hop1_Wq, hop1_Wk,
           h2_W0, h2_b0, h2_W1, h2_b1, h2_W2, h2_b2, h2_W3, h2_b3, hop2_Wq, hop2_Wk,
           h3_W0, h3_b0, h3_W1, h3_b1, h3_W2, h3_b2, h3_W3, h3_b3, hop3_Wq, hop3_Wk,
           train=0):
    per_model = [
        [h0_W0, h0_b0, h0_W1, h0_b1, h0_W2, h0_b2, h0_W3, h0_b3, hop0_Wq, hop0_Wk],
        [h1_W0, h1_b0, h1_W1, h1_b1, h1_W2, h1_b2, h1_W3, h1_b3, hop1_Wq, hop1_Wk],
        [h2_W0, h2_b0, h2_W1, h2_b1, h2_W2, h2_b2, h2_W3, h2_b3, hop2_Wq, hop2_Wk],
        [h3_W0, h3_b0, h3_W1, h3_b1, h3_W2, h3_b2, h3_W3, h3_b3, hop3_Wq, hop3_Wk],
    ]
    stacked = []
    for j in range(10):
        arrs = [per_model[i][j] for i in range(MHN)]
        st = jnp.stack(arrs)
        if st.ndim == 2:
            st = st[:, None, :]
        stacked.append(st)

    Y2 = Y_d[:, :, :, 0].reshape(B, MHN, 1, S)

    in_specs = [
        pl.BlockSpec((NB, 1, S, CIN), lambda i, b: (b, i, 0, 0)),
        pl.BlockSpec((NB, 1, 1, S), lambda i, b: (b, i, 0, 0)),
    ]
    for st in stacked:
        in_specs.append(
            pl.BlockSpec((1,) + st.shape[1:], lambda i, b: (i, 0, 0)))

    out_shape = [
        jax.ShapeDtypeStruct((B, MHN, NHEADS, A, S), jnp.float32),
        jax.ShapeDtypeStruct((B, MHN, NHEADS, A, S), jnp.float32),
        jax.ShapeDtypeStruct((MHN, 8, 128), jnp.float32),
    ]
    out_specs = [
        pl.BlockSpec((NB, 1, NHEADS, A, S), lambda i, b: (b, i, 0, 0, 0)),
        pl.BlockSpec((NB, 1, NHEADS, A, S), lambda i, b: (b, i, 0, 0, 0)),
        pl.BlockSpec((1, 8, 128), lambda i, b: (i, 0, 0)),
    ]

    ylow_k, yhigh_k, sc_k = pl.pallas_call(
        _body,
        grid=(MHN, B // NB),
        in_specs=in_specs,
        out_specs=out_specs,
        out_shape=out_shape,
        compiler_params=pltpu.CompilerParams(
            dimension_semantics=("arbitrary", "arbitrary")),
    )(X_d, Y2, *stacked)

    scores = sc_k[:, 0, 0] + jnp.asarray(train, jnp.float32) * 0.0
    y = jnp.transpose(Y_d[:, :, :, 0], (0, 2, 1))
    y_low = jnp.transpose(ylow_k, (2, 0, 3, 4, 1))
    y_high = jnp.transpose(yhigh_k, (2, 0, 3, 4, 1))
    return (scores, y, y_low, y_high)


# speedup vs baseline: 1.2552x; 1.0060x over previous
import functools

import numpy as np
import jax
import jax.numpy as jnp
from jax.experimental import pallas as pl
from jax.experimental.pallas import tpu as pltpu

MHN = 4
NHEADS = 4
CIN = 128
COUT = 128
B = 64
S = 256
A = 5
K = 20
ALPHAS = (0.05, 0.06, 0.08, 0.1, 0.12)


def _qpos(q):
    loc = q * (K - 1)
    f = int(np.floor(loc))
    return f, float(loc - f)


_LOW = [_qpos(a / 2.0) for a in ALPHAS]
_HIGH = [_qpos(1.0 - a + a / 2.0) for a in ALPHAS]


NB = 8


def _body(x_ref, y_ref, w0, c0, w1, c1, w2, c2, w3, c3, wq, wk,
          ylow_ref, yhigh_ref, sc_ref):
    bb = pl.program_id(1)
    xall = x_ref[:, 0].reshape(NB * S, CIN)

    h = jnp.maximum(jnp.dot(xall, w0[0], preferred_element_type=jnp.float32) + c0[0], 0.0)
    h = jnp.maximum(jnp.dot(h, w1[0], preferred_element_type=jnp.float32) + c1[0], 0.0)
    h = jnp.maximum(jnp.dot(h, w2[0], preferred_element_type=jnp.float32) + c2[0], 0.0)
    enc = jnp.dot(h, w3[0], preferred_element_type=jnp.float32) + c3[0]
    q = jnp.dot(enc, wq[0], preferred_element_type=jnp.float32)
    k = jnp.dot(enc, wk[0], preferred_element_type=jnp.float32)

    inf = jnp.float32(np.inf)
    total = jnp.float32(0.0)

    for j in range(NB):
        yrow = y_ref[j, 0]
        ybc = jnp.broadcast_to(yrow.reshape(S, 1), (S, S))
        qj = q[j * S:(j + 1) * S]
        kj = k[j * S:(j + 1) * S]
        curs = [jax.lax.dot_general(kj[:, hh * COUT:(hh + 1) * COUT],
                                    qj[:, hh * COUT:(hh + 1) * COUT],
                                    (((1,), (1,)), ((), ())),
                                    preferred_element_type=jnp.float32)
                for hh in range(NHEADS)]
        for _ in range(K):
            mxs = [jnp.max(c, axis=0, keepdims=True) for c in curs]
            curs = [jnp.where(c == m, -inf, c) for c, m in zip(curs, mxs)]
        sels = [c == -inf for c in curs]
        ylos = [jnp.where(s, ybc, inf) for s in sels]
        yhis = [jnp.where(s, ybc, -inf) for s in sels]
        m1s = [jnp.min(v, axis=0, keepdims=True) for v in ylos]
        ylos = [jnp.where(v == m, inf, v) for v, m in zip(ylos, m1s)]
        m2s = [jnp.min(v, axis=0, keepdims=True) for v in ylos]
        ylos = [jnp.where(v == m, inf, v) for v, m in zip(ylos, m2s)]
        m3s = [jnp.min(v, axis=0, keepdims=True) for v in ylos]
        b1s = [jnp.max(v, axis=0, keepdims=True) for v in yhis]
        yhis = [jnp.where(v == m, -inf, v) for v, m in zip(yhis, b1s)]
        b2s = [jnp.max(v, axis=0, keepdims=True) for v in yhis]
        yhis = [jnp.where(v == m, -inf, v) for v, m in zip(yhis, b2s)]
        b3s = [jnp.max(v, axis=0, keepdims=True) for v in yhis]

        for hh in range(NHEADS):
            m1, m2, m3 = m1s[hh], m2s[hh], m3s[hh]
            b1, b2, b3 = b1s[hh], b2s[hh], b3s[hh]
            lows = []
            for f, g in _LOW:
                lo_v, hi_v = (m1, m2) if f == 0 else (m2, m3)
                lows.append(lo_v * jnp.float32(1.0 - g) + hi_v * jnp.float32(g))
            highs = []
            for f, g in _HIGH:
                lo_v, hi_v = (b2, b1) if f == 18 else (b3, b2)
                highs.append(lo_v * jnp.float32(1.0 - g) + hi_v * jnp.float32(g))
            ylow_ref[j, 0, hh] = jnp.concatenate(lows, axis=0)
            yhigh_ref[j, 0, hh] = jnp.concatenate(highs, axis=0)
            for ai, a in enumerate(ALPHAS):
                ql = lows[ai]
                qh2 = highs[ai]
                c = jnp.float32(2.0 / a)
                pen = (jnp.abs(qh2 - ql)
                       + jnp.where(yrow < ql, (ql - yrow) * c, 0.0)
                       + jnp.where(yrow > qh2, (yrow - qh2) * c, 0.0))
                total = total + jnp.sum(pen)

    @pl.when(bb == 0)
    def _init():
        sc_ref[...] = jnp.zeros_like(sc_ref)

    sc_ref[...] += total * jnp.float32(1.0 / (NHEADS * A * B * S))


def kernel(X_d, Y_d,
           h0_W0, h0_b0, h0_W1, h0_b1, h0_W2, h0_b2, h0_W3, h0_b3, hop0_Wq, hop0_Wk,
           h1_W0, h1_b0, h1_W1, h1_b1, h1_W2, h1_b2, h1_W3, h1_b3, hop1_Wq, hop1_Wk,
           h2_W0, h2_b0, h2_W1, h2_b1, h2_W2, h2_b2, h2_W3, h2_b3, hop2_Wq, hop2_Wk,
           h3_W0, h3_b0, h3_W1, h3_b1, h3_W2, h3_b2, h3_W3, h3_b3, hop3_Wq, hop3_Wk,
           train=0):
    per_model = [
        [h0_W0, h0_b0, h0_W1, h0_b1, h0_W2, h0_b2, h0_W3, h0_b3, hop0_Wq, hop0_Wk],
        [h1_W0, h1_b0, h1_W1, h1_b1, h1_W2, h1_b2, h1_W3, h1_b3, hop1_Wq, hop1_Wk],
        [h2_W0, h2_b0, h2_W1, h2_b1, h2_W2, h2_b2, h2_W3, h2_b3, hop2_Wq, hop2_Wk],
        [h3_W0, h3_b0, h3_W1, h3_b1, h3_W2, h3_b2, h3_W3, h3_b3, hop3_Wq, hop3_Wk],
    ]
    stacked = []
    for j in range(10):
        arrs = [per_model[i][j] for i in range(MHN)]
        st = jnp.stack(arrs)
        if st.ndim == 2:
            st = st[:, None, :]
        stacked.append(st)

    Y2 = Y_d[:, :, :, 0].reshape(B, MHN, 1, S)

    in_specs = [
        pl.BlockSpec((NB, 1, S, CIN), lambda i, b: (b, i, 0, 0)),
        pl.BlockSpec((NB, 1, 1, S), lambda i, b: (b, i, 0, 0)),
    ]
    for st in stacked:
        in_specs.append(
            pl.BlockSpec((1,) + st.shape[1:], lambda i, b: (i, 0, 0)))

    out_shape = [
        jax.ShapeDtypeStruct((B, MHN, NHEADS, A, S), jnp.float32),
        jax.ShapeDtypeStruct((B, MHN, NHEADS, A, S), jnp.float32),
        jax.ShapeDtypeStruct((MHN, 8, 128), jnp.float32),
    ]
    out_specs = [
        pl.BlockSpec((NB, 1, NHEADS, A, S), lambda i, b: (b, i, 0, 0, 0)),
        pl.BlockSpec((NB, 1, NHEADS, A, S), lambda i, b: (b, i, 0, 0, 0)),
        pl.BlockSpec((1, 8, 128), lambda i, b: (i, 0, 0)),
    ]

    ylow_k, yhigh_k, sc_k = pl.pallas_call(
        _body,
        grid=(MHN, B // NB),
        in_specs=in_specs,
        out_specs=out_specs,
        out_shape=out_shape,
        compiler_params=pltpu.CompilerParams(
            dimension_semantics=("arbitrary", "arbitrary")),
    )(X_d, Y2, *stacked)

    scores = sc_k[:, 0, 0] + jnp.asarray(train, jnp.float32) * 0.0
    y = jnp.transpose(Y_d[:, :, :, 0], (0, 2, 1))
    y_low = jnp.transpose(ylow_k, (2, 0, 3, 4, 1))
    y_high = jnp.transpose(yhigh_k, (2, 0, 3, 4, 1))
    return (scores, y, y_low, y_high)


# speedup vs baseline: 1.8227x; 1.4522x over previous
import functools

import numpy as np
import jax
import jax.numpy as jnp
from jax.experimental import pallas as pl
from jax.experimental.pallas import tpu as pltpu

MHN = 4
NHEADS = 4
CIN = 128
COUT = 128
B = 64
S = 256
A = 5
K = 20
ALPHAS = (0.05, 0.06, 0.08, 0.1, 0.12)


def _qpos(q):
    loc = q * (K - 1)
    f = int(np.floor(loc))
    return f, float(loc - f)


_LOW = [_qpos(a / 2.0) for a in ALPHAS]
_HIGH = [_qpos(1.0 - a + a / 2.0) for a in ALPHAS]


NB = 8


def _merge4(a, b):
    e1 = jnp.maximum(a[0], b[3])
    e2 = jnp.maximum(a[1], b[2])
    e3 = jnp.maximum(a[2], b[1])
    e4 = jnp.maximum(a[3], b[0])
    f1 = jnp.maximum(e1, e3); f3 = jnp.minimum(e1, e3)
    f2 = jnp.maximum(e2, e4); f4 = jnp.minimum(e2, e4)
    g1 = jnp.maximum(f1, f2); g2 = jnp.minimum(f1, f2)
    g3 = jnp.maximum(f3, f4); g4 = jnp.minimum(f3, f4)
    return (g1, g2, g3, g4)


def _roll_sub(x, sh):
    return jnp.concatenate([x[sh:], x[:sh]], axis=0)


def _top4(cur):
    rows = [cur[r * 8:(r + 1) * 8] for r in range(S // 8)]
    l2 = []
    for p in range(len(rows) // 2):
        a, b = rows[2 * p], rows[2 * p + 1]
        l2.append((jnp.maximum(a, b), jnp.minimum(a, b)))
    l4 = []
    for p in range(len(l2) // 2):
        (a1, a2), (b1, b2) = l2[2 * p], l2[2 * p + 1]
        c1 = jnp.maximum(a1, b1); x1 = jnp.minimum(a1, b1)
        x2 = jnp.maximum(a2, b2); c4 = jnp.minimum(a2, b2)
        c2 = jnp.maximum(x1, x2); c3 = jnp.minimum(x1, x2)
        l4.append((c1, c2, c3, c4))
    lvl = l4
    while len(lvl) > 1:
        lvl = [_merge4(lvl[2 * p], lvl[2 * p + 1])
               for p in range(len(lvl) // 2)]
    a = lvl[0]
    for sh in (4, 2, 1):
        a = _merge4(a, tuple(_roll_sub(x, sh) for x in a))
    return a[3][0:1]


def _body(x_ref, y_ref, w0, c0, w1, c1, w2, c2, w3, c3, wq, wk,
          ylow_ref, yhigh_ref, sc_ref):
    bb = pl.program_id(1)
    xall = x_ref[:, 0].reshape(NB * S, CIN)

    h = jnp.maximum(jnp.dot(xall, w0[0], preferred_element_type=jnp.float32) + c0[0], 0.0)
    h = jnp.maximum(jnp.dot(h, w1[0], preferred_element_type=jnp.float32) + c1[0], 0.0)
    h = jnp.maximum(jnp.dot(h, w2[0], preferred_element_type=jnp.float32) + c2[0], 0.0)
    enc = jnp.dot(h, w3[0], preferred_element_type=jnp.float32) + c3[0]
    q = jnp.dot(enc, wq[0], preferred_element_type=jnp.float32)
    k = jnp.dot(enc, wk[0], preferred_element_type=jnp.float32)

    inf = jnp.float32(np.inf)
    total = jnp.float32(0.0)

    for j in range(NB):
        yrow = y_ref[j, 0]
        ybc = jnp.broadcast_to(yrow.reshape(S, 1), (S, S))
        qj = q[j * S:(j + 1) * S]
        kj = k[j * S:(j + 1) * S]
        curs = [jax.lax.dot_general(kj[:, hh * COUT:(hh + 1) * COUT],
                                    qj[:, hh * COUT:(hh + 1) * COUT],
                                    (((1,), (1,)), ((), ())),
                                    preferred_element_type=jnp.float32)
                for hh in range(NHEADS)]
        for _ in range(K // 4):
            t4s = [_top4(c) for c in curs]
            curs = [jnp.where(c >= t, -inf, c) for c, t in zip(curs, t4s)]
        sels = [c == -inf for c in curs]
        ylos = [jnp.where(s, ybc, inf) for s in sels]
        yhis = [jnp.where(s, ybc, -inf) for s in sels]
        m1s = [jnp.min(v, axis=0, keepdims=True) for v in ylos]
        ylos = [jnp.where(v == m, inf, v) for v, m in zip(ylos, m1s)]
        m2s = [jnp.min(v, axis=0, keepdims=True) for v in ylos]
        ylos = [jnp.where(v == m, inf, v) for v, m in zip(ylos, m2s)]
        m3s = [jnp.min(v, axis=0, keepdims=True) for v in ylos]
        b1s = [jnp.max(v, axis=0, keepdims=True) for v in yhis]
        yhis = [jnp.where(v == m, -inf, v) for v, m in zip(yhis, b1s)]
        b2s = [jnp.max(v, axis=0, keepdims=True) for v in yhis]
        yhis = [jnp.where(v == m, -inf, v) for v, m in zip(yhis, b2s)]
        b3s = [jnp.max(v, axis=0, keepdims=True) for v in yhis]

        for hh in range(NHEADS):
            m1, m2, m3 = m1s[hh], m2s[hh], m3s[hh]
            b1, b2, b3 = b1s[hh], b2s[hh], b3s[hh]
            lows = []
            for f, g in _LOW:
                lo_v, hi_v = (m1, m2) if f == 0 else (m2, m3)
                lows.append(lo_v * jnp.float32(1.0 - g) + hi_v * jnp.float32(g))
            highs = []
            for f, g in _HIGH:
                lo_v, hi_v = (b2, b1) if f == 18 else (b3, b2)
                highs.append(lo_v * jnp.float32(1.0 - g) + hi_v * jnp.float32(g))
            ylow_ref[j, 0, hh] = jnp.concatenate(lows, axis=0)
            yhigh_ref[j, 0, hh] = jnp.concatenate(highs, axis=0)
            for ai, a in enumerate(ALPHAS):
                ql = lows[ai]
                qh2 = highs[ai]
                c = jnp.float32(2.0 / a)
                pen = (jnp.abs(qh2 - ql)
                       + jnp.where(yrow < ql, (ql - yrow) * c, 0.0)
                       + jnp.where(yrow > qh2, (yrow - qh2) * c, 0.0))
                total = total + jnp.sum(pen)

    @pl.when(bb == 0)
    def _init():
        sc_ref[...] = jnp.zeros_like(sc_ref)

    sc_ref[...] += total * jnp.float32(1.0 / (NHEADS * A * B * S))


def kernel(X_d, Y_d,
           h0_W0, h0_b0, h0_W1, h0_b1, h0_W2, h0_b2, h0_W3, h0_b3, hop0_Wq, hop0_Wk,
           h1_W0, h1_b0, h1_W1, h1_b1, h1_W2, h1_b2, h1_W3, h1_b3, hop1_Wq, hop1_Wk,
           h2_W0, h2_b0, h2_W1, h2_b1, h2_W2, h2_b2, h2_W3, h2_b3, hop2_Wq, hop2_Wk,
           h3_W0, h3_b0, h3_W1, h3_b1, h3_W2, h3_b2, h3_W3, h3_b3, hop3_Wq, hop3_Wk,
           train=0):
    per_model = [
        [h0_W0, h0_b0, h0_W1, h0_b1, h0_W2, h0_b2, h0_W3, h0_b3, hop0_Wq, hop0_Wk],
        [h1_W0, h1_b0, h1_W1, h1_b1, h1_W2, h1_b2, h1_W3, h1_b3, hop1_Wq, hop1_Wk],
        [h2_W0, h2_b0, h2_W1, h2_b1, h2_W2, h2_b2, h2_W3, h2_b3, hop2_Wq, hop2_Wk],
        [h3_W0, h3_b0, h3_W1, h3_b1, h3_W2, h3_b2, h3_W3, h3_b3, hop3_Wq, hop3_Wk],
    ]
    stacked = []
    for j in range(10):
        arrs = [per_model[i][j] for i in range(MHN)]
        st = jnp.stack(arrs)
        if st.ndim == 2:
            st = st[:, None, :]
        stacked.append(st)

    Y2 = Y_d[:, :, :, 0].reshape(B, MHN, 1, S)

    in_specs = [
        pl.BlockSpec((NB, 1, S, CIN), lambda i, b: (b, i, 0, 0)),
        pl.BlockSpec((NB, 1, 1, S), lambda i, b: (b, i, 0, 0)),
    ]
    for st in stacked:
        in_specs.append(
            pl.BlockSpec((1,) + st.shape[1:], lambda i, b: (i, 0, 0)))

    out_shape = [
        jax.ShapeDtypeStruct((B, MHN, NHEADS, A, S), jnp.float32),
        jax.ShapeDtypeStruct((B, MHN, NHEADS, A, S), jnp.float32),
        jax.ShapeDtypeStruct((MHN, 8, 128), jnp.float32),
    ]
    out_specs = [
        pl.BlockSpec((NB, 1, NHEADS, A, S), lambda i, b: (b, i, 0, 0, 0)),
        pl.BlockSpec((NB, 1, NHEADS, A, S), lambda i, b: (b, i, 0, 0, 0)),
        pl.BlockSpec((1, 8, 128), lambda i, b: (i, 0, 0)),
    ]

    ylow_k, yhigh_k, sc_k = pl.pallas_call(
        _body,
        grid=(MHN, B // NB),
        in_specs=in_specs,
        out_specs=out_specs,
        out_shape=out_shape,
        compiler_params=pltpu.CompilerParams(
            dimension_semantics=("arbitrary", "arbitrary")),
    )(X_d, Y2, *stacked)

    scores = sc_k[:, 0, 0] + jnp.asarray(train, jnp.float32) * 0.0
    y = jnp.transpose(Y_d[:, :, :, 0], (0, 2, 1))
    y_low = jnp.transpose(ylow_k, (2, 0, 3, 4, 1))
    y_high = jnp.transpose(yhigh_k, (2, 0, 3, 4, 1))
    return (scores, y, y_low, y_high)
